# baseline (speedup 1.0000x reference)
import functools

import jax
import jax.numpy as jnp
from jax import lax
from jax.experimental import pallas as pl
from jax.experimental.pallas import tpu as pltpu
from jax.experimental.pallas import tpu_sc as plsc

VOCAB = 1000000
EMB = 32
BATCH = 16384
HIST = 50

NC = 2
NS = 16
NW = NC * NS
L = 16

BAT_PER_W = BATCH // NW
TB = 128
NTB = BAT_PER_W // TB
NUNIT = HIST * NTB
NBUF = 4


def _make_kernel():
  mesh = plsc.VectorSubcoreMesh(
      core_axis_name="c", subcore_axis_name="s", num_cores=NC, num_subcores=NS
  )

  @functools.partial(
      pl.kernel,
      out_type=jax.ShapeDtypeStruct((HIST, EMB, BATCH), jnp.float32),
      mesh=mesh,
      scratch_types=[
          pltpu.VMEM((BAT_PER_W, HIST), jnp.int32),
          pltpu.VMEM((HIST, BAT_PER_W), jnp.int32),
          [pltpu.VMEM((TB, EMB), jnp.float32) for _ in range(NBUF)],
          [pltpu.VMEM((EMB, TB), jnp.float32) for _ in range(NBUF)],
          [pltpu.SemaphoreType.DMA for _ in range(NBUF)],
          [pltpu.SemaphoreType.DMA for _ in range(NBUF)],
      ],
      compiler_params=pltpu.CompilerParams(
          use_tc_tiling_on_sc=False,
          needs_layout_passes=False,
          disable_bounds_checks=True,
      ),
  )
  def gather_kernel(x_hbm, w_hbm, out_hbm, idx_v, idxt_v, rows, tbufs,
                    gsems, wsems):
    wid = lax.axis_index("s") * NC + lax.axis_index("c")
    bat0 = wid * BAT_PER_W

    iota = lax.iota(jnp.int32, L)
    cvecs = [iota + (c * L) for c in range(TB // L)]
    evecs = [jnp.full((L,), e, jnp.int32) for e in range(EMB)]

    pltpu.sync_copy(x_hbm.at[pl.ds(bat0, BAT_PER_W)], idx_v)

    @pl.loop(0, HIST)
    def _(h):
      hvec = h + jnp.zeros((L,), jnp.int32)
      for c in range(BAT_PER_W // L):
        col = plsc.load_gather(idx_v, [iota + (c * L), hvec])
        idxt_v[h, pl.ds(c * L, L)] = col

    def start_g(u, b):
      h = u // NTB
      tb = lax.rem(u, NTB)
      pltpu.async_copy(
          w_hbm.at[idxt_v.at[h, pl.ds(tb * TB, TB)]], rows[b], gsems[b]
      )

    def wait_g(b):
      pltpu.make_async_copy(w_hbm.at[pl.ds(0, TB)], rows[b], gsems[b]).wait()

    def transpose(b):
      for e in range(EMB):
        for c in range(TB // L):
          v = plsc.load_gather(rows[b], [cvecs[c], evecs[e]])
          tbufs[b][e, pl.ds(c * L, L)] = v

    def _dst(u):
      h = u // NTB
      tb = lax.rem(u, NTB)
      return out_hbm.at[h, :, pl.ds(bat0 + tb * TB, TB)]

    def start_w(u, b):
      pltpu.async_copy(tbufs[b], _dst(u), wsems[b])

    def wait_w(u, b):
      pltpu.make_async_copy(tbufs[b], _dst(u), wsems[b]).wait()

    start_g(0, 0)
    start_g(1, 1)
    start_g(2, 2)
    wait_g(0)
    transpose(0)
    start_w(0, 0)
    start_g(3, 3)
    wait_g(1)
    transpose(1)
    start_w(1, 1)

    @pl.loop(4, NUNIT - NUNIT % NBUF, step=NBUF)
    def _(u0):
      for j in range(NBUF):
        u = u0 + j
        b = j
        b2 = (j + 2) % NBUF
        wait_w(u - NBUF, b)
        start_g(u, b)
        wait_g(b2)
        transpose(b2)
        start_w(u - 2, b2)

    wait_g(2)
    transpose(2)
    start_w(198, 2)
    wait_g(3)
    transpose(3)
    start_w(199, 3)
    wait_w(196, 0)
    wait_w(197, 1)
    wait_w(198, 2)
    wait_w(199, 3)

  return gather_kernel


_kernel_call = _make_kernel()


@jax.jit
def kernel(x, W):
  out_t = _kernel_call(x.astype(jnp.int32), W)
  return jnp.transpose(out_t, (2, 0, 1))


# speedup vs baseline: 1.4396x; 1.4396x over previous
import functools

import jax
import jax.numpy as jnp
from jax import lax
from jax.experimental import pallas as pl
from jax.experimental.pallas import tpu as pltpu
from jax.experimental.pallas import tpu_sc as plsc

VOCAB = 1000000
EMB = 32
BATCH = 16384
HIST = 50

NC = 2
NS = 16
NW = NC * NS
L = 16

BAT_PER_W = BATCH // NW
TB = 128
NTB = BAT_PER_W // TB
NUNIT = HIST * NTB
NBUF = 4


def _make_kernel():
  mesh = plsc.VectorSubcoreMesh(
      core_axis_name="c", subcore_axis_name="s", num_cores=NC, num_subcores=NS
  )

  @functools.partial(
      pl.kernel,
      out_type=jax.ShapeDtypeStruct((HIST, EMB, BATCH), jnp.float32),
      mesh=mesh,
      scratch_types=[
          pltpu.VMEM((BAT_PER_W, HIST), jnp.int32),
          pltpu.VMEM((HIST, BAT_PER_W), jnp.int32),
          [pltpu.VMEM((TB, EMB), jnp.float32) for _ in range(NBUF)],
          [pltpu.VMEM((EMB, TB + 1), jnp.float32) for _ in range(NBUF)],
          [pltpu.SemaphoreType.DMA for _ in range(NBUF)],
          [pltpu.SemaphoreType.DMA for _ in range(NBUF)],
      ],
      compiler_params=pltpu.CompilerParams(
          use_tc_tiling_on_sc=False,
          needs_layout_passes=False,
          disable_bounds_checks=True,
      ),
  )
  def gather_kernel(x_hbm, w_hbm, out_hbm, idx_v, idxt_v, rows, tbufs,
                    gsems, wsems):
    wid = lax.axis_index("s") * NC + lax.axis_index("c")
    bat0 = wid * BAT_PER_W

    iota = lax.iota(jnp.int32, L)
    e0vec = iota
    e1vec = iota + L

    pltpu.sync_copy(x_hbm.at[pl.ds(bat0, BAT_PER_W)], idx_v)

    @pl.loop(0, HIST)
    def _(h):
      hvec = h + jnp.zeros((L,), jnp.int32)
      for c in range(BAT_PER_W // L):
        col = plsc.load_gather(idx_v, [iota + (c * L), hvec])
        idxt_v[h, pl.ds(c * L, L)] = col

    def start_g(u, b):
      h = u // NTB
      tb = lax.rem(u, NTB)
      pltpu.async_copy(
          w_hbm.at[idxt_v.at[h, pl.ds(tb * TB, TB)]], rows[b], gsems[b]
      )

    def wait_g(b):
      pltpu.make_async_copy(w_hbm.at[pl.ds(0, TB)], rows[b], gsems[b]).wait()

    def transpose(b):
      for r in range(TB):
        rvec = jnp.full((L,), r, jnp.int32)
        lo = rows[b][r, pl.ds(0, L)]
        hi = rows[b][r, pl.ds(L, L)]
        plsc.store_scatter(tbufs[b], [e0vec, rvec], lo)
        plsc.store_scatter(tbufs[b], [e1vec, rvec], hi)

    def _dst(u):
      h = u // NTB
      tb = lax.rem(u, NTB)
      return out_hbm.at[h, :, pl.ds(bat0 + tb * TB, TB)]

    def start_w(u, b):
      pltpu.async_copy(tbufs[b].at[:, pl.ds(0, TB)], _dst(u), wsems[b])

    def wait_w(u, b):
      pltpu.make_async_copy(
          tbufs[b].at[:, pl.ds(0, TB)], _dst(u), wsems[b]
      ).wait()

    start_g(0, 0)
    start_g(1, 1)
    start_g(2, 2)
    wait_g(0)
    transpose(0)
    start_w(0, 0)
    start_g(3, 3)
    wait_g(1)
    transpose(1)
    start_w(1, 1)

    @pl.loop(4, NUNIT - NUNIT % NBUF, step=NBUF)
    def _(u0):
      for j in range(NBUF):
        u = u0 + j
        b = j
        b2 = (j + 2) % NBUF
        wait_w(u - NBUF, b)
        start_g(u, b)
        wait_g(b2)
        transpose(b2)
        start_w(u - 2, b2)

    wait_g(2)
    transpose(2)
    start_w(198, 2)
    wait_g(3)
    transpose(3)
    start_w(199, 3)
    wait_w(196, 0)
    wait_w(197, 1)
    wait_w(198, 2)
    wait_w(199, 3)

  return gather_kernel


_kernel_call = _make_kernel()


@jax.jit
def kernel(x, W):
  out_t = _kernel_call(x.astype(jnp.int32), W)
  return jnp.transpose(out_t, (2, 0, 1))
